# baseline (speedup 1.0000x reference)
import jax
import jax.numpy as jnp
from jax.experimental import pallas as pl

UNITS = 64
CUTOFF = 5.0


def _vector_to_skewtensor(v):
    zero = jnp.zeros_like(v[:, 0])
    t = jnp.stack([zero, -v[:, 2], v[:, 1], v[:, 2], zero, -v[:, 0], -v[:, 1], v[:, 0], zero], axis=-1)
    return t.reshape(-1, 3, 3)


def _vector_to_symtensor(v):
    t = v[:, :, None] * v[:, None, :]
    tr = (t[:, 0, 0] + t[:, 1, 1] + t[:, 2, 2]) / 3.0
    eye = jnp.eye(3, dtype=v.dtype)
    return 0.5 * (t + jnp.swapaxes(t, -2, -1)) - tr[:, None, None] * eye[None]


def _cosine_cutoff(r, cutoff):
    return 0.5 * (jnp.cos(r * jnp.pi / cutoff) + 1.0) * (r < cutoff)


def _layernorm(x, g, b, eps=1e-5):
    mu = jnp.mean(x, -1, keepdims=True)
    var = jnp.mean((x - mu) ** 2, -1, keepdims=True)
    return (x - mu) / jnp.sqrt(var + eps) * g + b


def _copy_kernel(x_ref, o_ref):
    o_ref[...] = x_ref[...]


def kernel(z, edge_index, edge_weight, edge_vec, edge_attr, emb, Wd1, bd1, Wd2, bd2, Wd3, bd3, Wemb2, bemb2, Wt0, Wt1, Wt2, Ws0, bs0, Ws1, bs1, ln_g, ln_b):
    num_nodes = z.shape[0]
    x = jnp.take(emb, z, axis=0)
    C = _cosine_cutoff(edge_weight, CUTOFF)
    W1 = (edge_attr @ Wd1.T + bd1) * C[:, None]
    W2 = (edge_attr @ Wd2.T + bd2) * C[:, None]
    W3 = (edge_attr @ Wd3.T + bd3) * C[:, None]
    evn = edge_vec / jnp.maximum(jnp.linalg.norm(edge_vec, axis=1, keepdims=True), 1e-6)
    src, dst = edge_index[0], edge_index[1]
    vi = x[src]
    vj = x[dst]
    Zij = jnp.concatenate([vi, vj], axis=-1) @ Wemb2.T + bemb2
    ea = jnp.stack([W1, W2, W3], axis=1) * Zij[:, None, :]
    A_base = _vector_to_skewtensor(evn)[:, None, :, :]
    S_base = _vector_to_symtensor(evn)[:, None, :, :]
    eye_scalar = jnp.ones((1, 1, 1, 1), dtype=evn.dtype)
    fI, fA, fS = ea[:, 0, :], ea[:, 1, :], ea[:, 2, :]
    I_ij = jnp.transpose(fI[..., None, None] * eye_scalar, (0, 2, 3, 1))
    A_ij = jnp.transpose(fA[..., None, None] * A_base, (0, 2, 3, 1))
    S_ij = jnp.transpose(fS[..., None, None] * S_base, (0, 2, 3, 1))
    I_t = jax.ops.segment_sum(I_ij, dst, num_segments=num_nodes)
    A = jax.ops.segment_sum(A_ij, dst, num_segments=num_nodes)
    S = jax.ops.segment_sum(S_ij, dst, num_segments=num_nodes)
    eye = jnp.eye(3, dtype=I_t.dtype)
    I_exp = I_t[:, 0, 0, :][:, None, None, :] * eye[None, :, :, None]
    X = I_exp + A + S
    norm = jnp.sum(X ** 2, axis=(-3, -2))
    norm = _layernorm(norm, ln_g, ln_b)
    I_lin = (I_exp.reshape(-1, UNITS) @ Wt0.T).reshape(I_exp.shape)
    A_lin = (A.reshape(-1, UNITS) @ Wt1.T).reshape(A.shape)
    S_lin = (S.reshape(-1, UNITS) @ Wt2.T).reshape(S.shape)
    h = jax.nn.silu(norm @ Ws0.T + bs0)
    h = jax.nn.silu(h @ Ws1.T + bs1)
    h = h.reshape(num_nodes, 3, UNITS)
    Xout = h[:, 0][:, None, None, :] * I_lin + h[:, 1][:, None, None, :] * A_lin + h[:, 2][:, None, None, :] * S_lin
    Xout2 = Xout.reshape(num_nodes, 9 * UNITS)
    out = pl.pallas_call(
        _copy_kernel,
        out_shape=jax.ShapeDtypeStruct(Xout2.shape, Xout2.dtype),
        grid=(10,),
        in_specs=[pl.BlockSpec((1000, 9 * UNITS), lambda i: (i, 0))],
        out_specs=pl.BlockSpec((1000, 9 * UNITS), lambda i: (i, 0)),
    )(Xout2)
    return out.reshape(num_nodes, 3, 3, UNITS)


# speedup vs baseline: 22.3086x; 22.3086x over previous
import math

import jax
import jax.numpy as jnp
from jax import lax
from jax.experimental import pallas as pl
from jax.experimental.pallas import tpu as pltpu
from jax.experimental.pallas import tpu_sc as plsc

F32 = jnp.float32
_NC, _NS = 2, 16
_NW = _NC * _NS
_CH = 128
_U = 64
_MW = 640
_HI = lax.Precision.HIGHEST


def _node_pre_body(z_ref, emb_ref, waT_ref, wbT_ref, bb_ref, xcat_ref):
    z = z_ref[...]
    nt = emb_ref.shape[0]
    ids = lax.broadcasted_iota(jnp.int32, (z.shape[0], nt), 1)
    onehot = (ids == z).astype(F32)
    x = jnp.dot(onehot, emb_ref[...], precision=_HI)
    xa = jnp.dot(x, waT_ref[...], precision=_HI)
    xb = jnp.dot(x, wbT_ref[...], precision=_HI) + bb_ref[...]
    xcat_ref[...] = jnp.concatenate([xa, xb], axis=1)


def _node_pre(z2, emb, waT, wbT, bb):
    n = z2.shape[0]
    return pl.pallas_call(
        _node_pre_body,
        out_shape=jax.ShapeDtypeStruct((n, 2 * _U), F32),
    )(z2, emb, waT, wbT, bb)


def _gather_body(src_hbm, dst_hbm, xcat_hbm, o_hbm,
                 sidx, didx, bufa, bufb, sema, semb):
    e = src_hbm.shape[0]
    nch = e // _CH
    wid = lax.axis_index("s") * _NC + lax.axis_index("c")

    def body(i, carry):
        c = wid + _NW * i

        @pl.when(c < nch)
        def _():
            base = c * _CH
            pltpu.sync_copy(src_hbm.at[pl.ds(base, _CH)], sidx)
            pltpu.sync_copy(dst_hbm.at[pl.ds(base, _CH)], didx)
            da = pltpu.async_copy(xcat_hbm.at[sidx], bufa, sema)
            db = pltpu.async_copy(xcat_hbm.at[didx], bufb, semb)
            da.wait()
            db.wait()
            pltpu.sync_copy(bufa, o_hbm.at[pl.ds(base, _CH), pl.ds(0, 2 * _U)])
            pltpu.sync_copy(bufb, o_hbm.at[pl.ds(base, _CH),
                                           pl.ds(2 * _U, 2 * _U)])
        return carry

    lax.fori_loop(0, (nch + _NW - 1) // _NW, body, 0)


def _gather(src, dst, xcat):
    e = src.shape[0]
    mesh = plsc.VectorSubcoreMesh(core_axis_name="c", subcore_axis_name="s",
                                  num_cores=_NC, num_subcores=_NS)
    f = pl.kernel(
        _gather_body,
        out_type=jax.ShapeDtypeStruct((e, 4 * _U), F32),
        mesh=mesh,
        scratch_types=[
            pltpu.VMEM((_CH,), jnp.int32),
            pltpu.VMEM((_CH,), jnp.int32),
            pltpu.VMEM((_CH, 2 * _U), F32),
            pltpu.VMEM((_CH, 2 * _U), F32),
            pltpu.SemaphoreType.DMA,
            pltpu.SemaphoreType.DMA,
        ],
    )
    return f(src, dst, xcat)


def _edge_body(attr_ref, w_ref, vec_ref, x2_ref, wdT_ref, bd_ref, m_ref):
    w = w_ref[...]
    cutw = 0.5 * (jnp.cos(w * (math.pi / 5.0)) + 1.0)
    cutw = jnp.where(w < 5.0, cutw, 0.0)
    x2 = x2_ref[...]
    z = x2[:, 0:_U] + x2[:, 3 * _U:4 * _U]
    attr = attr_ref[...]
    g1 = (jnp.dot(attr, wdT_ref[0], precision=_HI) + bd_ref[:, 0, :]) * cutw
    g2 = (jnp.dot(attr, wdT_ref[1], precision=_HI) + bd_ref[:, 1, :]) * cutw
    g3 = (jnp.dot(attr, wdT_ref[2], precision=_HI) + bd_ref[:, 2, :]) * cutw
    f1 = g1 * z
    f2 = g2 * z
    f3 = g3 * z
    vx = vec_ref[:, 0:1]
    vy = vec_ref[:, 1:2]
    vz = vec_ref[:, 2:3]
    inv = 1.0 / jnp.maximum(jnp.sqrt(vx * vx + vy * vy + vz * vz), 1e-6)
    ex = vx * inv
    ey = vy * inv
    ez = vz * inv
    tr3 = (ex * ex + ey * ey + ez * ez) * (1.0 / 3.0)
    m_ref[...] = jnp.concatenate(
        [f1, f2 * ex, f2 * ey, f2 * ez,
         f3 * (ex * ex - tr3), f3 * (ey * ey - tr3),
         f3 * (ex * ey), f3 * (ex * ez), f3 * (ey * ez),
         jnp.zeros_like(f1)], axis=1)


def _edge_msgs(attr, w2, vec, x2, wdT3, bd3):
    e = attr.shape[0]
    be = 2000
    return pl.pallas_call(
        _edge_body,
        grid=(e // be,),
        in_specs=[
            pl.BlockSpec((be, attr.shape[1]), lambda i: (i, 0)),
            pl.BlockSpec((be, 1), lambda i: (i, 0)),
            pl.BlockSpec((be, 3), lambda i: (i, 0)),
            pl.BlockSpec((be, 4 * _U), lambda i: (i, 0)),
            pl.BlockSpec((3, attr.shape[1], _U), lambda i: (0, 0, 0)),
            pl.BlockSpec((1, 3, _U), lambda i: (0, 0, 0)),
        ],
        out_specs=pl.BlockSpec((be, _MW), lambda i: (i, 0)),
        out_shape=jax.ShapeDtypeStruct((e, _MW), F32),
    )(attr, w2, vec, x2, wdT3, bd3)


def _scatter_body(m_hbm, dst_hbm, acc_hbm, shacc, idxv, buf, zbuf):
    e = dst_hbm.shape[0]
    n = acc_hbm.shape[0]
    nch = e // _CH
    cid = lax.axis_index("c")
    sid = lax.axis_index("s")
    row0 = sid * 624
    z16 = jnp.zeros((16,), F32)

    def zrow(r, carry):
        for j in range(8):
            zbuf[r, pl.ds(j * 16, 16)] = z16
        return carry

    lax.fori_loop(0, 208, zrow, 0)

    for p in range(3):
        chunk = jnp.minimum(cid * 3 + p, 4)
        coff = chunk * _CH

        for zi in range(3):
            pltpu.sync_copy(zbuf, shacc.at[pl.ds(row0 + zi * 208, 208)])

        @pl.when(sid == _NS - 1)
        def _():
            pltpu.sync_copy(zbuf.at[pl.ds(0, 16)],
                            shacc.at[pl.ds(n - 16, 16)])

        plsc.subcore_barrier()

        def body(i, carry):
            c = sid + _NS * i

            @pl.when(c < nch)
            def _():
                base = c * _CH
                pltpu.sync_copy(dst_hbm.at[pl.ds(base, _CH)], idxv)
                pltpu.sync_copy(m_hbm.at[pl.ds(base, _CH), pl.ds(coff, _CH)],
                                buf)
                pltpu.sync_copy(buf, shacc.at[idxv], add=True)
            return carry

        lax.fori_loop(0, (nch + _NS - 1) // _NS, body, 0)
        plsc.subcore_barrier()

        pltpu.sync_copy(shacc.at[pl.ds(row0, 624)],
                        acc_hbm.at[pl.ds(row0, 624), pl.ds(coff, _CH)])

        @pl.when(sid == _NS - 1)
        def _():
            pltpu.sync_copy(shacc.at[pl.ds(n - 16, 16)],
                            acc_hbm.at[pl.ds(n - 16, 16), pl.ds(coff, _CH)])

        plsc.subcore_barrier()


def _segment_sum(m, dst, n):
    mesh = plsc.VectorSubcoreMesh(core_axis_name="c", subcore_axis_name="s",
                                  num_cores=_NC, num_subcores=_NS)
    f = pl.kernel(
        _scatter_body,
        out_type=jax.ShapeDtypeStruct((n, _MW), F32),
        mesh=mesh,
        scratch_types=[
            pltpu.VMEM_SHARED((n, _CH), F32),
            pltpu.VMEM((_CH,), jnp.int32),
            pltpu.VMEM((_CH, _CH), F32),
            pltpu.VMEM((208, _CH), F32),
        ],
    )
    return f(m, dst)


def _silu(t):
    return t * (1.0 / (1.0 + jnp.exp(-t)))


def _node_post_body(acc_ref, wt0_ref, wt1_ref, wt2_ref, ws0_ref, bs0_ref,
                    ws1_ref, bs1_ref, lg_ref, lb_ref, o_ref):
    a = acc_ref[...]
    i_ = a[:, 0 * _U:1 * _U]
    px = a[:, 1 * _U:2 * _U]
    py = a[:, 2 * _U:3 * _U]
    pz = a[:, 3 * _U:4 * _U]
    qxx = a[:, 4 * _U:5 * _U]
    qyy = a[:, 5 * _U:6 * _U]
    qxy = a[:, 6 * _U:7 * _U]
    qxz = a[:, 7 * _U:8 * _U]
    qyz = a[:, 8 * _U:9 * _U]
    qzz = -(qxx + qyy)

    x00 = i_ + qxx
    x11 = i_ + qyy
    x22 = i_ + qzz
    norm = (x00 * x00 + x11 * x11 + x22 * x22
            + 2.0 * (qxy * qxy + pz * pz + qxz * qxz + py * py
                     + qyz * qyz + px * px))
    mu = jnp.mean(norm, axis=-1, keepdims=True)
    var = jnp.mean((norm - mu) ** 2, axis=-1, keepdims=True)
    ln = (norm - mu) / jnp.sqrt(var + 1e-5) * lg_ref[...] + lb_ref[...]

    h = _silu(jnp.dot(ln, ws0_ref[...], precision=_HI) + bs0_ref[...])
    hh = _silu(jnp.dot(h, ws1_ref[...], precision=_HI) + bs1_ref[...])
    h0 = hh[:, 0:_U]
    h1 = hh[:, _U:2 * _U]
    h2 = hh[:, 2 * _U:3 * _U]

    wt0 = wt0_ref[...]
    wt1 = wt1_ref[...]
    wt2 = wt2_ref[...]
    il = jnp.dot(i_, wt0, precision=_HI)
    plx = jnp.dot(px, wt1, precision=_HI)
    ply = jnp.dot(py, wt1, precision=_HI)
    plz = jnp.dot(pz, wt1, precision=_HI)
    qlxx = jnp.dot(qxx, wt2, precision=_HI)
    qlyy = jnp.dot(qyy, wt2, precision=_HI)
    qlxy = jnp.dot(qxy, wt2, precision=_HI)
    qlxz = jnp.dot(qxz, wt2, precision=_HI)
    qlyz = jnp.dot(qyz, wt2, precision=_HI)
    qlzz = -(qlxx + qlyy)

    hil = h0 * il
    o_ref[:, 0, 0, :] = hil + h2 * qlxx
    o_ref[:, 1, 1, :] = hil + h2 * qlyy
    o_ref[:, 2, 2, :] = hil + h2 * qlzz
    o_ref[:, 0, 1, :] = h2 * qlxy - h1 * plz
    o_ref[:, 1, 0, :] = h2 * qlxy + h1 * plz
    o_ref[:, 0, 2, :] = h2 * qlxz + h1 * ply
    o_ref[:, 2, 0, :] = h2 * qlxz - h1 * ply
    o_ref[:, 1, 2, :] = h2 * qlyz - h1 * plx
    o_ref[:, 2, 1, :] = h2 * qlyz + h1 * plx


def _node_post(acc, wt0T, wt1T, wt2T, ws0T, bs0_2, ws1T, bs1_2, lg2, lb2):
    n = acc.shape[0]
    bn = 1000
    return pl.pallas_call(
        _node_post_body,
        grid=(n // bn,),
        in_specs=[
            pl.BlockSpec((bn, _MW), lambda i: (i, 0)),
            pl.BlockSpec((_U, _U), lambda i: (0, 0)),
            pl.BlockSpec((_U, _U), lambda i: (0, 0)),
            pl.BlockSpec((_U, _U), lambda i: (0, 0)),
            pl.BlockSpec((_U, 2 * _U), lambda i: (0, 0)),
            pl.BlockSpec((1, 2 * _U), lambda i: (0, 0)),
            pl.BlockSpec((2 * _U, 3 * _U), lambda i: (0, 0)),
            pl.BlockSpec((1, 3 * _U), lambda i: (0, 0)),
            pl.BlockSpec((1, _U), lambda i: (0, 0)),
            pl.BlockSpec((1, _U), lambda i: (0, 0)),
        ],
        out_specs=pl.BlockSpec((bn, 3, 3, _U), lambda i: (i, 0, 0, 0)),
        out_shape=jax.ShapeDtypeStruct((n, 3, 3, _U), F32),
    )(acc, wt0T, wt1T, wt2T, ws0T, bs0_2, ws1T, bs1_2, lg2, lb2)


def kernel(z, edge_index, edge_weight, edge_vec, edge_attr, emb, Wd1, bd1,
           Wd2, bd2, Wd3, bd3, Wemb2, bemb2, Wt0, Wt1, Wt2, Ws0, bs0, Ws1,
           bs1, ln_g, ln_b):
    n = z.shape[0]
    e = edge_index.shape[1]
    src = edge_index[0]
    dst = edge_index[1]

    waT = Wemb2[:, :_U].T
    wbT = Wemb2[:, _U:].T
    bb = bemb2.reshape(1, _U)
    xcat = _node_pre(z.reshape(n, 1), emb, waT, wbT, bb)

    x2 = _gather(src, dst, xcat)

    wdT3 = jnp.stack([Wd1.T, Wd2.T, Wd3.T], axis=0)
    bd3_ = jnp.stack([bd1, bd2, bd3], axis=0).reshape(1, 3, _U)
    m = _edge_msgs(edge_attr, edge_weight.reshape(e, 1), edge_vec, x2,
                   wdT3, bd3_)

    acc = _segment_sum(m, dst, n)

    return _node_post(acc, Wt0.T, Wt1.T, Wt2.T, Ws0.T,
                      bs0.reshape(1, 2 * _U), Ws1.T,
                      bs1.reshape(1, 3 * _U), ln_g.reshape(1, _U),
                      ln_b.reshape(1, _U))


# speedup vs baseline: 25.1435x; 1.1271x over previous
import math

import jax
import jax.numpy as jnp
from jax import lax
from jax.experimental import pallas as pl
from jax.experimental.pallas import tpu as pltpu
from jax.experimental.pallas import tpu_sc as plsc

F32 = jnp.float32
_NC, _NS = 2, 16
_NW = _NC * _NS
_CH = 128
_U = 64
_MW = 640
_HI = lax.Precision.HIGHEST


def _node_pre_body(z_ref, emb_ref, waT_ref, wbT_ref, bb_ref, xcat_ref):
    z = z_ref[...]
    nt = emb_ref.shape[0]
    ids = lax.broadcasted_iota(jnp.int32, (z.shape[0], nt), 1)
    onehot = (ids == z).astype(F32)
    x = jnp.dot(onehot, emb_ref[...], precision=_HI)
    xa = jnp.dot(x, waT_ref[...], precision=_HI)
    xb = jnp.dot(x, wbT_ref[...], precision=_HI) + bb_ref[...]
    xcat_ref[...] = jnp.concatenate([xa, xb], axis=1)


def _node_pre(z2, emb, waT, wbT, bb):
    n = z2.shape[0]
    return pl.pallas_call(
        _node_pre_body,
        out_shape=jax.ShapeDtypeStruct((n, 2 * _U), F32),
    )(z2, emb, waT, wbT, bb)


def _gather_body(src_hbm, dst_hbm, xcat_hbm, o_hbm,
                 sidx, didx, bufa, bufb, sema, semb):
    e = src_hbm.shape[0]
    nch = e // _CH
    wid = lax.axis_index("s") * _NC + lax.axis_index("c")

    def body(i, carry):
        c = wid + _NW * i

        @pl.when(c < nch)
        def _():
            base = c * _CH
            pltpu.sync_copy(src_hbm.at[pl.ds(base, _CH)], sidx)
            pltpu.sync_copy(dst_hbm.at[pl.ds(base, _CH)], didx)
            da = pltpu.async_copy(xcat_hbm.at[sidx], bufa, sema)
            db = pltpu.async_copy(xcat_hbm.at[didx], bufb, semb)
            da.wait()
            db.wait()
            pltpu.sync_copy(bufa, o_hbm.at[pl.ds(base, _CH), pl.ds(0, 2 * _U)])
            pltpu.sync_copy(bufb, o_hbm.at[pl.ds(base, _CH),
                                           pl.ds(2 * _U, 2 * _U)])
        return carry

    lax.fori_loop(0, (nch + _NW - 1) // _NW, body, 0)


def _gather(src, dst, xcat):
    e = src.shape[0]
    mesh = plsc.VectorSubcoreMesh(core_axis_name="c", subcore_axis_name="s",
                                  num_cores=_NC, num_subcores=_NS)
    f = pl.kernel(
        _gather_body,
        out_type=jax.ShapeDtypeStruct((e, 4 * _U), F32),
        mesh=mesh,
        scratch_types=[
            pltpu.VMEM((_CH,), jnp.int32),
            pltpu.VMEM((_CH,), jnp.int32),
            pltpu.VMEM((_CH, 2 * _U), F32),
            pltpu.VMEM((_CH, 2 * _U), F32),
            pltpu.SemaphoreType.DMA,
            pltpu.SemaphoreType.DMA,
        ],
    )
    return f(src, dst, xcat)


def _edge_body(attr_ref, w_ref, vec_ref, x2_ref, wdT_ref, bd_ref, m_ref):
    w = w_ref[...]
    cutw = 0.5 * (jnp.cos(w * (math.pi / 5.0)) + 1.0)
    cutw = jnp.where(w < 5.0, cutw, 0.0)
    x2 = x2_ref[...]
    z = x2[:, 0:_U] + x2[:, 3 * _U:4 * _U]
    attr = attr_ref[...]
    g1 = (jnp.dot(attr, wdT_ref[0], precision=_HI) + bd_ref[:, 0, :]) * cutw
    g2 = (jnp.dot(attr, wdT_ref[1], precision=_HI) + bd_ref[:, 1, :]) * cutw
    g3 = (jnp.dot(attr, wdT_ref[2], precision=_HI) + bd_ref[:, 2, :]) * cutw
    f1 = g1 * z
    f2 = g2 * z
    f3 = g3 * z
    vx = vec_ref[:, 0:1]
    vy = vec_ref[:, 1:2]
    vz = vec_ref[:, 2:3]
    inv = 1.0 / jnp.maximum(jnp.sqrt(vx * vx + vy * vy + vz * vz), 1e-6)
    ex = vx * inv
    ey = vy * inv
    ez = vz * inv
    tr3 = (ex * ex + ey * ey + ez * ez) * (1.0 / 3.0)
    m_ref[...] = jnp.concatenate(
        [f1, f2 * ex, f2 * ey, f2 * ez,
         f3 * (ex * ex - tr3), f3 * (ey * ey - tr3),
         f3 * (ex * ey), f3 * (ex * ez), f3 * (ey * ez),
         jnp.zeros_like(f1)], axis=1)


def _edge_msgs(attr, w2, vec, x2, wdT3, bd3):
    e = attr.shape[0]
    be = 2000
    return pl.pallas_call(
        _edge_body,
        grid=(e // be,),
        in_specs=[
            pl.BlockSpec((be, attr.shape[1]), lambda i: (i, 0)),
            pl.BlockSpec((be, 1), lambda i: (i, 0)),
            pl.BlockSpec((be, 3), lambda i: (i, 0)),
            pl.BlockSpec((be, 4 * _U), lambda i: (i, 0)),
            pl.BlockSpec((3, attr.shape[1], _U), lambda i: (0, 0, 0)),
            pl.BlockSpec((1, 3, _U), lambda i: (0, 0, 0)),
        ],
        out_specs=pl.BlockSpec((be, _MW), lambda i: (i, 0)),
        out_shape=jax.ShapeDtypeStruct((e, _MW), F32),
    )(attr, w2, vec, x2, wdT3, bd3)


def _scatter_body(m_hbm, dst_hbm, acc_hbm, shacc, idxv, buf, zbuf):
    e = dst_hbm.shape[0]
    n = acc_hbm.shape[1]
    nch = e // _CH
    cid = lax.axis_index("c")
    sid = lax.axis_index("s")
    row0 = sid * 624
    z16 = jnp.zeros((16,), F32)

    def zrow(r, carry):
        for j in range(8):
            zbuf[r, pl.ds(j * 16, 16)] = z16
        return carry

    lax.fori_loop(0, 208, zrow, 0)

    for p in range(5):
        coff = p * _CH

        for zi in range(3):
            pltpu.sync_copy(zbuf, shacc.at[pl.ds(row0 + zi * 208, 208)])

        @pl.when(sid == _NS - 1)
        def _():
            pltpu.sync_copy(zbuf.at[pl.ds(0, 16)],
                            shacc.at[pl.ds(n - 16, 16)])

        plsc.subcore_barrier()

        def body(i, carry):
            g = _NW * i + sid * _NC + cid

            @pl.when(g < nch)
            def _():
                base = g * _CH
                pltpu.sync_copy(dst_hbm.at[pl.ds(base, _CH)], idxv)
                pltpu.sync_copy(m_hbm.at[pl.ds(base, _CH), pl.ds(coff, _CH)],
                                buf)
                pltpu.sync_copy(buf, shacc.at[idxv], add=True)
            return carry

        lax.fori_loop(0, (nch + _NW - 1) // _NW, body, 0)
        plsc.subcore_barrier()

        pltpu.sync_copy(shacc.at[pl.ds(row0, 624)],
                        acc_hbm.at[cid, pl.ds(row0, 624), pl.ds(coff, _CH)])

        @pl.when(sid == _NS - 1)
        def _():
            pltpu.sync_copy(shacc.at[pl.ds(n - 16, 16)],
                            acc_hbm.at[cid, pl.ds(n - 16, 16),
                                       pl.ds(coff, _CH)])

        plsc.subcore_barrier()


def _segment_sum(m, dst, n):
    mesh = plsc.VectorSubcoreMesh(core_axis_name="c", subcore_axis_name="s",
                                  num_cores=_NC, num_subcores=_NS)
    f = pl.kernel(
        _scatter_body,
        out_type=jax.ShapeDtypeStruct((_NC, n, _MW), F32),
        mesh=mesh,
        scratch_types=[
            pltpu.VMEM_SHARED((n, _CH), F32),
            pltpu.VMEM((_CH,), jnp.int32),
            pltpu.VMEM((_CH, _CH), F32),
            pltpu.VMEM((208, _CH), F32),
        ],
    )
    return f(m, dst)


def _silu(t):
    return t * (1.0 / (1.0 + jnp.exp(-t)))


def _node_post_body(acc0_ref, acc1_ref, wt0_ref, wt1_ref, wt2_ref, ws0_ref,
                    bs0_ref, ws1_ref, bs1_ref, lg_ref, lb_ref, o_ref):
    a = (acc0_ref[0] + acc0_ref[1]) + (acc1_ref[0] + acc1_ref[1])
    i_ = a[:, 0 * _U:1 * _U]
    px = a[:, 1 * _U:2 * _U]
    py = a[:, 2 * _U:3 * _U]
    pz = a[:, 3 * _U:4 * _U]
    qxx = a[:, 4 * _U:5 * _U]
    qyy = a[:, 5 * _U:6 * _U]
    qxy = a[:, 6 * _U:7 * _U]
    qxz = a[:, 7 * _U:8 * _U]
    qyz = a[:, 8 * _U:9 * _U]
    qzz = -(qxx + qyy)

    x00 = i_ + qxx
    x11 = i_ + qyy
    x22 = i_ + qzz
    norm = (x00 * x00 + x11 * x11 + x22 * x22
            + 2.0 * (qxy * qxy + pz * pz + qxz * qxz + py * py
                     + qyz * qyz + px * px))
    mu = jnp.mean(norm, axis=-1, keepdims=True)
    var = jnp.mean((norm - mu) ** 2, axis=-1, keepdims=True)
    ln = (norm - mu) / jnp.sqrt(var + 1e-5) * lg_ref[...] + lb_ref[...]

    h = _silu(jnp.dot(ln, ws0_ref[...], precision=_HI) + bs0_ref[...])
    hh = _silu(jnp.dot(h, ws1_ref[...], precision=_HI) + bs1_ref[...])
    h0 = hh[:, 0:_U]
    h1 = hh[:, _U:2 * _U]
    h2 = hh[:, 2 * _U:3 * _U]

    wt0 = wt0_ref[...]
    wt1 = wt1_ref[...]
    wt2 = wt2_ref[...]
    il = jnp.dot(i_, wt0, precision=_HI)
    plx = jnp.dot(px, wt1, precision=_HI)
    ply = jnp.dot(py, wt1, precision=_HI)
    plz = jnp.dot(pz, wt1, precision=_HI)
    qlxx = jnp.dot(qxx, wt2, precision=_HI)
    qlyy = jnp.dot(qyy, wt2, precision=_HI)
    qlxy = jnp.dot(qxy, wt2, precision=_HI)
    qlxz = jnp.dot(qxz, wt2, precision=_HI)
    qlyz = jnp.dot(qyz, wt2, precision=_HI)
    qlzz = -(qlxx + qlyy)

    hil = h0 * il
    o_ref[:, 0, 0, :] = hil + h2 * qlxx
    o_ref[:, 1, 1, :] = hil + h2 * qlyy
    o_ref[:, 2, 2, :] = hil + h2 * qlzz
    o_ref[:, 0, 1, :] = h2 * qlxy - h1 * plz
    o_ref[:, 1, 0, :] = h2 * qlxy + h1 * plz
    o_ref[:, 0, 2, :] = h2 * qlxz + h1 * ply
    o_ref[:, 2, 0, :] = h2 * qlxz - h1 * ply
    o_ref[:, 1, 2, :] = h2 * qlyz - h1 * plx
    o_ref[:, 2, 1, :] = h2 * qlyz + h1 * plx


def _node_post(acc0, acc1, wt0T, wt1T, wt2T, ws0T, bs0_2, ws1T, bs1_2, lg2,
               lb2):
    n = acc0.shape[1]
    bn = 1000
    return pl.pallas_call(
        _node_post_body,
        grid=(n // bn,),
        in_specs=[
            pl.BlockSpec((_NC, bn, _MW), lambda i: (0, i, 0)),
            pl.BlockSpec((_NC, bn, _MW), lambda i: (0, i, 0)),
            pl.BlockSpec((_U, _U), lambda i: (0, 0)),
            pl.BlockSpec((_U, _U), lambda i: (0, 0)),
            pl.BlockSpec((_U, _U), lambda i: (0, 0)),
            pl.BlockSpec((_U, 2 * _U), lambda i: (0, 0)),
            pl.BlockSpec((1, 2 * _U), lambda i: (0, 0)),
            pl.BlockSpec((2 * _U, 3 * _U), lambda i: (0, 0)),
            pl.BlockSpec((1, 3 * _U), lambda i: (0, 0)),
            pl.BlockSpec((1, _U), lambda i: (0, 0)),
            pl.BlockSpec((1, _U), lambda i: (0, 0)),
        ],
        out_specs=pl.BlockSpec((bn, 3, 3, _U), lambda i: (i, 0, 0, 0)),
        out_shape=jax.ShapeDtypeStruct((n, 3, 3, _U), F32),
    )(acc0, acc1, wt0T, wt1T, wt2T, ws0T, bs0_2, ws1T, bs1_2, lg2, lb2)


def kernel(z, edge_index, edge_weight, edge_vec, edge_attr, emb, Wd1, bd1,
           Wd2, bd2, Wd3, bd3, Wemb2, bemb2, Wt0, Wt1, Wt2, Ws0, bs0, Ws1,
           bs1, ln_g, ln_b):
    n = z.shape[0]
    e = edge_index.shape[1]
    src = edge_index[0]
    dst = edge_index[1]

    waT = Wemb2[:, :_U].T
    wbT = Wemb2[:, _U:].T
    bb = bemb2.reshape(1, _U)
    xcat = _node_pre(z.reshape(n, 1), emb, waT, wbT, bb)

    wdT3 = jnp.stack([Wd1.T, Wd2.T, Wd3.T], axis=0)
    bd3_ = jnp.stack([bd1, bd2, bd3], axis=0).reshape(1, 3, _U)
    w2 = edge_weight.reshape(e, 1)

    h = e // 2
    parts = []
    x2s = [_gather(src[k * h:(k + 1) * h], dst[k * h:(k + 1) * h], xcat)
           for k in range(2)]
    for k in range(2):
        m = _edge_msgs(edge_attr[k * h:(k + 1) * h], w2[k * h:(k + 1) * h],
                       edge_vec[k * h:(k + 1) * h], x2s[k], wdT3, bd3_)
        parts.append(_segment_sum(m, dst[k * h:(k + 1) * h], n))

    return _node_post(parts[0], parts[1], Wt0.T, Wt1.T, Wt2.T, Ws0.T,
                      bs0.reshape(1, 2 * _U), Ws1.T,
                      bs1.reshape(1, 3 * _U), ln_g.reshape(1, _U),
                      ln_b.reshape(1, _U))
